# Initial kernel scaffold; baseline (speedup 1.0000x reference)
import numpy as np
import jax
import jax.numpy as jnp
from jax import lax
from jax.experimental import pallas as pl
from jax.experimental.pallas import tpu as pltpu
from jax.experimental.pallas import tpu_sc as plsc

N_MEM_C = 1000000
B_C = 4096
TWOB_C = 8192
ALPHA_C = np.float32(0.5)

_MOM = np.float32(N_MEM_C) / (np.float32(N_MEM_C) + np.float32(B_C))
_ONE_MINUS_MOM = np.float32(1.0) - _MOM


RED_G = 32
RED_ROWS = B_C // RED_G


def _reduce_body(qii_ref, qij_ref, qji_ref, c_ref, acc_ref):
    step = pl.program_id(0)

    @pl.when(step == 0)
    def _init():
        acc_ref[0, 0] = jnp.float32(0.0)

    acc_ref[0, 0] += jnp.sum(qij_ref[...]) + jnp.sum(qji_ref[...])

    @pl.when(step == RED_G - 1)
    def _finish():
        b = jnp.float32(B_C)
        nf = jnp.float32(N_MEM_C)
        alpha = jnp.float32(ALPHA_C)
        sii = jnp.sum(qii_ref[...])
        xi = alpha * sii + (jnp.float32(1.0) - alpha) * (acc_ref[0, 0] / (4.0 * b))
        omega = b
        c_ref[0, 0] = jnp.float32(_MOM) * nf * (xi / omega)


def _reduce_c(qii2d, qij, qji):
    return pl.pallas_call(
        _reduce_body,
        grid=(RED_G,),
        in_specs=[
            pl.BlockSpec((32, 128), lambda i: (0, 0)),
            pl.BlockSpec((RED_ROWS, TWOB_C), lambda i: (i, 0)),
            pl.BlockSpec((RED_ROWS, TWOB_C), lambda i: (i, 0)),
        ],
        out_specs=pl.BlockSpec(memory_space=pltpu.SMEM),
        out_shape=jax.ShapeDtypeStruct((1, 1), jnp.float32),
        scratch_shapes=[pltpu.SMEM((1, 1), jnp.float32)],
    )(qii2d, qij, qji)


NC = 2
NS = 16
NW = NC * NS
L = 16
CHUNK = 31264
TAIL = N_MEM_C - (NW - 1) * CHUNK
NVEC = B_C // L


def _sc_body(idx_hbm, sinv_hbm, c_hbm, out_hbm, chunk_v, idx_v, nv_v, c_v):
    cid = lax.axis_index("c")
    sid = lax.axis_index("s")
    wid = sid * NC + cid
    base = pl.multiple_of(wid * CHUNK, 8)

    pltpu.sync_copy(idx_hbm, idx_v)
    pltpu.sync_copy(c_hbm, c_v)

    def run(size):
        pltpu.sync_copy(sinv_hbm.at[pl.ds(base, size)], chunk_v.at[pl.ds(0, size)])
        lo = base
        hi = base + size
        cvec = c_v[...]

        def gather_body(j, carry):
            off = pl.multiple_of(j * L, L)
            idxv = idx_v[pl.ds(off, L)]
            inb = (idxv >= lo) & (idxv < hi)
            loc = jnp.where(inb, idxv - lo, 0)
            g = plsc.load_gather(chunk_v, [loc], mask=inb)
            nv_v[pl.ds(off, L)] = g * jnp.float32(_ONE_MINUS_MOM) + cvec
            return carry

        lax.fori_loop(0, NVEC, gather_body, 0)

        def scatter_body(j, carry):
            off = pl.multiple_of(j * L, L)
            idxv = idx_v[pl.ds(off, L)]
            inb = (idxv >= lo) & (idxv < hi)
            loc = jnp.where(inb, idxv - lo, 0)
            plsc.store_scatter(chunk_v, [loc], nv_v[pl.ds(off, L)], mask=inb)
            return carry

        lax.fori_loop(0, NVEC, scatter_body, 0)
        pltpu.sync_copy(chunk_v.at[pl.ds(0, size)], out_hbm.at[pl.ds(base, size)])

    @pl.when(wid < NW - 1)
    def _main():
        run(CHUNK)

    @pl.when(wid == NW - 1)
    def _tail():
        run(TAIL)


def _sc_scatter(idx32, s_inv, c16):
    mesh = plsc.VectorSubcoreMesh(core_axis_name="c", subcore_axis_name="s")
    f = pl.kernel(
        _sc_body,
        out_type=jax.ShapeDtypeStruct((N_MEM_C,), jnp.float32),
        mesh=mesh,
        scratch_types=[
            pltpu.VMEM((CHUNK,), jnp.float32),
            pltpu.VMEM((B_C,), jnp.int32),
            pltpu.VMEM((B_C,), jnp.float32),
            pltpu.VMEM((L,), jnp.float32),
        ],
    )
    return f(idx32, s_inv, c16)


def kernel(qii, qij, qji, feats_idx, s_inv):
    qii2d = qii.reshape(32, 128)
    c = _reduce_c(qii2d, qij, qji)
    c16 = jnp.broadcast_to(c.reshape(()), (L,))
    idx32 = feats_idx.astype(jnp.int32)
    return _sc_scatter(idx32, s_inv, c16)


# speedup vs baseline: 1.0932x; 1.0932x over previous
import numpy as np
import jax
import jax.numpy as jnp
from jax import lax
from jax.experimental import pallas as pl
from jax.experimental.pallas import tpu as pltpu
from jax.experimental.pallas import tpu_sc as plsc

N_MEM_C = 1000000
B_C = 4096
TWOB_C = 8192
ALPHA_C = np.float32(0.5)

_MOM = np.float32(N_MEM_C) / (np.float32(N_MEM_C) + np.float32(B_C))
_ONE_MINUS_MOM = np.float32(1.0) - _MOM


RED_G = 32
RED_ROWS = B_C // RED_G


def _reduce_body(qii_ref, qij_ref, qji_ref, c_ref, acc_ref):
    step = pl.program_id(0)

    @pl.when(step == 0)
    def _init():
        acc_ref[0, 0] = jnp.float32(0.0)

    acc_ref[0, 0] += jnp.sum(qij_ref[...]) + jnp.sum(qji_ref[...])

    @pl.when(step == RED_G - 1)
    def _finish():
        b = jnp.float32(B_C)
        nf = jnp.float32(N_MEM_C)
        alpha = jnp.float32(ALPHA_C)
        sii = jnp.sum(qii_ref[...])
        xi = alpha * sii + (jnp.float32(1.0) - alpha) * (acc_ref[0, 0] / (4.0 * b))
        omega = b
        c_ref[0, 0] = jnp.float32(_MOM) * nf * (xi / omega)


def _reduce_c(qii2d, qij, qji):
    return pl.pallas_call(
        _reduce_body,
        grid=(RED_G,),
        in_specs=[
            pl.BlockSpec((32, 128), lambda i: (0, 0)),
            pl.BlockSpec((RED_ROWS, TWOB_C), lambda i: (i, 0)),
            pl.BlockSpec((RED_ROWS, TWOB_C), lambda i: (i, 0)),
        ],
        out_specs=pl.BlockSpec(memory_space=pltpu.SMEM),
        out_shape=jax.ShapeDtypeStruct((1, 1), jnp.float32),
        scratch_shapes=[pltpu.SMEM((1, 1), jnp.float32)],
    )(qii2d, qij, qji)


NC = 2
NS = 16
NW = NC * NS
L = 16
CHUNK = 31264
TAIL = N_MEM_C - (NW - 1) * CHUNK
NVEC = B_C // L


def _sc_body(idx_hbm, sinv_hbm, c_hbm, out_hbm, chunk_v, idx_v, nv_v, c_v):
    cid = lax.axis_index("c")
    sid = lax.axis_index("s")
    wid = sid * NC + cid
    base = pl.multiple_of(wid * CHUNK, 8)

    pltpu.sync_copy(idx_hbm, idx_v)
    pltpu.sync_copy(c_hbm, c_v)

    def run(size):
        pltpu.sync_copy(sinv_hbm.at[pl.ds(base, size)], chunk_v.at[pl.ds(0, size)])
        lo = base
        hi = base + size
        cvec = c_v[...]

        def gather_body(j, carry):
            off = pl.multiple_of(j * L, L)
            idxv = idx_v[pl.ds(off, L)]
            inb = (idxv >= lo) & (idxv < hi)
            loc = jnp.where(inb, idxv - lo, 0)
            g = plsc.load_gather(chunk_v, [loc], mask=inb)
            nv_v[pl.ds(off, L)] = g * jnp.float32(_ONE_MINUS_MOM) + cvec
            return carry

        lax.fori_loop(0, NVEC, gather_body, 0)

        def scatter_body(j, carry):
            off = pl.multiple_of(j * L, L)
            idxv = idx_v[pl.ds(off, L)]
            inb = (idxv >= lo) & (idxv < hi)
            loc = jnp.where(inb, idxv - lo, 0)
            plsc.store_scatter(chunk_v, [loc], nv_v[pl.ds(off, L)], mask=inb)
            return carry

        lax.fori_loop(0, NVEC, scatter_body, 0)
        pltpu.sync_copy(chunk_v.at[pl.ds(0, size)], out_hbm.at[pl.ds(base, size)])

    @pl.when(wid < NW - 1)
    def _main():
        run(CHUNK)

    @pl.when(wid == NW - 1)
    def _tail():
        run(TAIL)


def _sc_scatter(idx32, s_inv, c16):
    mesh = plsc.VectorSubcoreMesh(core_axis_name="c", subcore_axis_name="s")
    f = pl.kernel(
        _sc_body,
        out_type=jax.ShapeDtypeStruct((N_MEM_C,), jnp.float32),
        mesh=mesh,
        scratch_types=[
            pltpu.VMEM((CHUNK,), jnp.float32),
            pltpu.VMEM((B_C,), jnp.int32),
            pltpu.VMEM((B_C,), jnp.float32),
            pltpu.VMEM((L,), jnp.float32),
        ],
        compiler_params=pltpu.CompilerParams(needs_layout_passes=False),
    )
    return f(idx32, s_inv, c16)


def kernel(qii, qij, qji, feats_idx, s_inv):
    qii2d = qii.reshape(32, 128)
    c = _reduce_c(qii2d, qij, qji)
    c16 = jnp.broadcast_to(c.reshape(()), (L,))
    idx32 = feats_idx.astype(jnp.int32)
    return _sc_scatter(idx32, s_inv, c16)


# speedup vs baseline: 1.0963x; 1.0028x over previous
import numpy as np
import jax
import jax.numpy as jnp
from jax import lax
from jax.experimental import pallas as pl
from jax.experimental.pallas import tpu as pltpu
from jax.experimental.pallas import tpu_sc as plsc

N_MEM_C = 1000000
B_C = 4096
TWOB_C = 8192
ALPHA_C = np.float32(0.5)

_MOM = np.float32(N_MEM_C) / (np.float32(N_MEM_C) + np.float32(B_C))
_ONE_MINUS_MOM = np.float32(1.0) - _MOM


RED_G = 16
RED_ROWS = B_C // RED_G


def _reduce_body(qii_ref, qij_ref, qji_ref, c_ref, acc_ref):
    step = pl.program_id(0)

    @pl.when(step == 0)
    def _init():
        acc_ref[0, 0] = jnp.float32(0.0)

    acc_ref[0, 0] += jnp.sum(qij_ref[...]) + jnp.sum(qji_ref[...])

    @pl.when(step == RED_G - 1)
    def _finish():
        b = jnp.float32(B_C)
        nf = jnp.float32(N_MEM_C)
        alpha = jnp.float32(ALPHA_C)
        sii = jnp.sum(qii_ref[...])
        xi = alpha * sii + (jnp.float32(1.0) - alpha) * (acc_ref[0, 0] / (4.0 * b))
        omega = b
        c_ref[0, 0] = jnp.float32(_MOM) * nf * (xi / omega)


def _reduce_c(qii2d, qij, qji):
    return pl.pallas_call(
        _reduce_body,
        grid=(RED_G,),
        in_specs=[
            pl.BlockSpec((32, 128), lambda i: (0, 0)),
            pl.BlockSpec((RED_ROWS, TWOB_C), lambda i: (i, 0)),
            pl.BlockSpec((RED_ROWS, TWOB_C), lambda i: (i, 0)),
        ],
        out_specs=pl.BlockSpec(memory_space=pltpu.SMEM),
        out_shape=jax.ShapeDtypeStruct((1, 1), jnp.float32),
        scratch_shapes=[pltpu.SMEM((1, 1), jnp.float32)],
    )(qii2d, qij, qji)


NC = 2
NS = 16
NW = NC * NS
L = 16
CHUNK = 31264
TAIL = N_MEM_C - (NW - 1) * CHUNK
NVEC = B_C // L


def _sc_body(idx_hbm, sinv_hbm, c_hbm, out_hbm, chunk_v, idx_v, nv_v, c_v):
    cid = lax.axis_index("c")
    sid = lax.axis_index("s")
    wid = sid * NC + cid
    base = pl.multiple_of(wid * CHUNK, 8)

    pltpu.sync_copy(idx_hbm, idx_v)
    pltpu.sync_copy(c_hbm, c_v)

    def run(size):
        pltpu.sync_copy(sinv_hbm.at[pl.ds(base, size)], chunk_v.at[pl.ds(0, size)])
        lo = base
        hi = base + size
        cvec = c_v[...]

        def gather_body(j, carry):
            off = pl.multiple_of(j * L, L)
            idxv = idx_v[pl.ds(off, L)]
            inb = (idxv >= lo) & (idxv < hi)
            loc = jnp.where(inb, idxv - lo, 0)
            g = plsc.load_gather(chunk_v, [loc], mask=inb)
            nv_v[pl.ds(off, L)] = g * jnp.float32(_ONE_MINUS_MOM) + cvec
            return carry

        lax.fori_loop(0, NVEC, gather_body, 0)

        def scatter_body(j, carry):
            off = pl.multiple_of(j * L, L)
            idxv = idx_v[pl.ds(off, L)]
            inb = (idxv >= lo) & (idxv < hi)
            loc = jnp.where(inb, idxv - lo, 0)
            plsc.store_scatter(chunk_v, [loc], nv_v[pl.ds(off, L)], mask=inb)
            return carry

        lax.fori_loop(0, NVEC, scatter_body, 0)
        pltpu.sync_copy(chunk_v.at[pl.ds(0, size)], out_hbm.at[pl.ds(base, size)])

    @pl.when(wid < NW - 1)
    def _main():
        run(CHUNK)

    @pl.when(wid == NW - 1)
    def _tail():
        run(TAIL)


def _sc_scatter(idx32, s_inv, c16):
    mesh = plsc.VectorSubcoreMesh(core_axis_name="c", subcore_axis_name="s")
    f = pl.kernel(
        _sc_body,
        out_type=jax.ShapeDtypeStruct((N_MEM_C,), jnp.float32),
        mesh=mesh,
        scratch_types=[
            pltpu.VMEM((CHUNK,), jnp.float32),
            pltpu.VMEM((B_C,), jnp.int32),
            pltpu.VMEM((B_C,), jnp.float32),
            pltpu.VMEM((L,), jnp.float32),
        ],
        compiler_params=pltpu.CompilerParams(needs_layout_passes=False),
    )
    return f(idx32, s_inv, c16)


def kernel(qii, qij, qji, feats_idx, s_inv):
    qii2d = qii.reshape(32, 128)
    c = _reduce_c(qii2d, qij, qji)
    c16 = jnp.broadcast_to(c.reshape(()), (L,))
    idx32 = feats_idx.astype(jnp.int32)
    return _sc_scatter(idx32, s_inv, c16)


# speedup vs baseline: 1.1457x; 1.0451x over previous
import numpy as np
import jax
import jax.numpy as jnp
from jax import lax
from jax.experimental import pallas as pl
from jax.experimental.pallas import tpu as pltpu
from jax.experimental.pallas import tpu_sc as plsc

N_MEM_C = 1000000
B_C = 4096
TWOB_C = 8192
ALPHA_C = np.float32(0.5)

_MOM = np.float32(N_MEM_C) / (np.float32(N_MEM_C) + np.float32(B_C))
_ONE_MINUS_MOM = np.float32(1.0) - _MOM


RED_G = 16
RED_ROWS = B_C // RED_G


def _reduce_body(qii_ref, qij_ref, qji_ref, sums_ref, acc_ref):
    step = pl.program_id(0)

    @pl.when(step == 0)
    def _init():
        acc_ref[0, 0] = jnp.float32(0.0)

    acc_ref[0, 0] += jnp.sum(qij_ref[...]) + jnp.sum(qji_ref[...])

    @pl.when(step == RED_G - 1)
    def _finish():
        s = acc_ref[0, 0]
        sii = jnp.sum(qii_ref[...])
        for j in range(16):
            sums_ref[0, j] = s
            sums_ref[1, j] = sii


def _reduce_tc(qii2d, qij, qji):
    return pl.pallas_call(
        _reduce_body,
        grid=(RED_G,),
        in_specs=[
            pl.BlockSpec((32, 128), lambda i: (0, 0)),
            pl.BlockSpec((RED_ROWS, TWOB_C), lambda i: (i, 0)),
            pl.BlockSpec((RED_ROWS, TWOB_C), lambda i: (i, 0)),
        ],
        out_specs=pl.BlockSpec(memory_space=pltpu.SMEM),
        out_shape=jax.ShapeDtypeStruct((2, 16), jnp.float32),
        scratch_shapes=[pltpu.SMEM((1, 1), jnp.float32)],
    )(qii2d, qij, qji)


NC = 2
NS = 16
NW = NC * NS
L = 16
PERW = B_C // NW

_SC_PARAMS = pltpu.CompilerParams(needs_layout_passes=False)


def _front_body(idx_hbm, sinv_hbm, pg_hbm, idx_v, pg_v, sem):
    cid = lax.axis_index("c")
    sid = lax.axis_index("s")
    wid = sid * NC + cid
    base = pl.multiple_of(wid * PERW, 8)
    pltpu.sync_copy(idx_hbm.at[pl.ds(base, PERW)], idx_v)
    pltpu.async_copy(sinv_hbm.at[idx_v], pg_v, sem).wait()
    for j in range(PERW // L):
        pg_v[pl.ds(j * L, L)] = pg_v[pl.ds(j * L, L)] * jnp.float32(_ONE_MINUS_MOM)
    pltpu.sync_copy(pg_v, pg_hbm.at[pl.ds(base, PERW)])


def _sc_front(idx32, s_inv):
    mesh = plsc.VectorSubcoreMesh(core_axis_name="c", subcore_axis_name="s")
    f = pl.kernel(
        _front_body,
        out_type=jax.ShapeDtypeStruct((B_C,), jnp.float32),
        mesh=mesh,
        scratch_types=[
            pltpu.VMEM((PERW,), jnp.int32),
            pltpu.VMEM((PERW,), jnp.float32),
            pltpu.SemaphoreType.DMA,
        ],
        compiler_params=_SC_PARAMS,
    )
    return f(idx32, s_inv)


def _finish_body(idx_hbm, pg_hbm, sums_hbm, out_hbm, idx_v, pg_v, sums_v, sem):
    cid = lax.axis_index("c")
    sid = lax.axis_index("s")
    wid = sid * NC + cid
    base = pl.multiple_of(wid * PERW, 8)
    pltpu.sync_copy(idx_hbm.at[pl.ds(base, PERW)], idx_v)
    pltpu.sync_copy(pg_hbm.at[pl.ds(base, PERW)], pg_v)
    pltpu.sync_copy(sums_hbm, sums_v)

    s = sums_v[pl.ds(0, L)]
    sii = sums_v[pl.ds(L, L)]
    b = jnp.float32(B_C)
    nf = jnp.float32(N_MEM_C)
    alpha = jnp.float32(ALPHA_C)
    xi = alpha * sii + (jnp.float32(1.0) - alpha) * (s / (4.0 * b))
    cvec = jnp.float32(_MOM) * nf * (xi / b)

    for j in range(PERW // L):
        pg_v[pl.ds(j * L, L)] = pg_v[pl.ds(j * L, L)] + cvec
    pltpu.async_copy(pg_v, out_hbm.at[idx_v], sem).wait()


def _sc_finish(idx32, pg, sums_flat, out_ref):
    mesh = plsc.VectorSubcoreMesh(core_axis_name="c", subcore_axis_name="s")
    f = pl.kernel(
        _finish_body,
        out_type=(),
        mesh=mesh,
        scratch_types=[
            pltpu.VMEM((PERW,), jnp.int32),
            pltpu.VMEM((PERW,), jnp.float32),
            pltpu.VMEM((2 * L,), jnp.float32),
            pltpu.SemaphoreType.DMA,
        ],
        compiler_params=_SC_PARAMS,
    )
    f(idx32, pg, sums_flat, out_ref)


def kernel(qii, qij, qji, feats_idx, s_inv):
    idx32 = feats_idx.astype(jnp.int32)
    pg = _sc_front(idx32, s_inv)
    qii2d = qii.reshape(32, 128)
    sums = _reduce_tc(qii2d, qij, qji)
    out_ref = jax.new_ref(s_inv)
    _sc_finish(idx32, pg, sums.reshape(2 * L), out_ref)
    return out_ref[...]
